# baseline (speedup 1.0000x reference)
import functools

import jax
import jax.numpy as jnp
from jax import lax
from jax.experimental import pallas as pl
from jax.experimental.pallas import tpu as pltpu
from jax.experimental.pallas import tpu_sc as plsc

N_NODES = 10000
N_PAD = 10112
D_FEAT = 128
NC = 2
NS = 16
NW = NC * NS
L = 16
CHUNK = 112
NBUF = 3


def _sc_partials(x, pack, n_chunks):
    rows_per_tile = N_PAD // NS

    mesh = plsc.VectorSubcoreMesh(core_axis_name="c", subcore_axis_name="s")

    @functools.partial(
        pl.kernel,
        out_type=jax.ShapeDtypeStruct((NC, N_PAD, D_FEAT), jnp.float32),
        mesh=mesh,
        scratch_types=[
            pltpu.VMEM((NBUF, 8, CHUNK), jnp.float32),
            pltpu.VMEM((CHUNK + L,), jnp.float32),
            pltpu.VMEM((CHUNK + L,), jnp.float32),
            pltpu.VMEM((CHUNK + L,), jnp.float32),
            pltpu.VMEM((CHUNK,), jnp.int32),
            pltpu.VMEM((CHUNK,), jnp.int32),
            pltpu.VMEM((CHUNK,), jnp.int32),
            pltpu.VMEM((CHUNK,), jnp.int32),
            pltpu.VMEM((CHUNK,), jnp.int32),
            pltpu.VMEM((CHUNK,), jnp.int32),
            pltpu.VMEM((NBUF, CHUNK, D_FEAT), jnp.float32),
            pltpu.VMEM_SHARED((N_PAD, D_FEAT), jnp.float32),
            pltpu.SemaphoreType.DMA,
            pltpu.SemaphoreType.DMA,
            pltpu.SemaphoreType.DMA,
            pltpu.SemaphoreType.DMA,
            pltpu.SemaphoreType.DMA,
            pltpu.SemaphoreType.DMA,
        ],
    )
    def k(x_hbm, pack_hbm, out_hbm, pack_v, b0, b1, b2, ri0, ri1, ri2,
          ci0, ci1, ci2, rows_v, acc, sg0, sg1, sg2, sp0, sp1, sp2):
        sgs = [sg0, sg1, sg2]
        sps = [sp0, sp1, sp2]
        bvs = [b0, b1, b2]
        ris = [ri0, ri1, ri2]
        cis = [ci0, ci1, ci2]
        cid = lax.axis_index("c")
        sid = lax.axis_index("s")
        wid = sid * NC + cid
        chunk0 = wid * n_chunks

        zeros = jnp.zeros((L,), jnp.float32)
        def zrow(r, _):
            for g in range(D_FEAT // L):
                rows_v[0, r, pl.ds(g * L, L)] = zeros
            return 0
        lax.fori_loop(0, CHUNK, zrow, 0)
        r0 = sid * rows_per_tile
        for piece in range(5):
            pltpu.sync_copy(rows_v.at[0],
                            acc.at[pl.ds(r0 + piece * CHUNK, CHUNK)])
        rem = rows_per_tile - 5 * CHUNK
        pltpu.sync_copy(rows_v.at[0, pl.ds(0, rem)],
                        acc.at[pl.ds(r0 + 5 * CHUNK, rem)])
        plsc.subcore_barrier()

        def pack_start(t, s):
            pltpu.async_copy(pack_hbm.at[chunk0 + t], pack_v.at[s], sps[s])

        def pack_wait(s):
            pltpu.make_async_copy(pack_hbm.at[chunk0], pack_v.at[s],
                                  sps[s]).wait()

        def unpack(s):
            for g in range(CHUNK // L):
                sl = pl.ds(g * L, L)
                ris[s][sl] = pack_v[s, 0, sl].astype(jnp.int32)
                cis[s][sl] = pack_v[s, 1, sl].astype(jnp.int32)
                bvs[s][sl] = pack_v[s, 2, sl]

        def gather_start(s):
            h = CHUNK // 2
            pltpu.async_copy(x_hbm.at[cis[s].at[pl.ds(0, h)]],
                             rows_v.at[s, pl.ds(0, h)], sgs[s])
            pltpu.async_copy(x_hbm.at[cis[s].at[pl.ds(h, h)]],
                             rows_v.at[s, pl.ds(h, h)], sgs[s])

        def gather_wait(s):
            pltpu.make_async_copy(x_hbm.at[cis[s]],
                                  rows_v.at[s], sgs[s]).wait()

        def scatter_sync(s):
            pltpu.sync_copy(rows_v.at[s], acc.at[ris[s]], add=True)

        def compute(s):
            bv = bvs[s]
            for g in range(CHUNK // L):
                bv[pl.ds(g * L, L)] = jnp.exp(-bv[pl.ds(g * L, L)])
            def edge_body(e, _):
                b = bv[pl.ds(e, L)][0]
                for g in range(D_FEAT // L):
                    rows_v[s, e, pl.ds(g * L, L)] = (
                        rows_v[s, e, pl.ds(g * L, L)] * b)
                return 0
            lax.fori_loop(0, CHUNK, edge_body, 0, unroll=8)

        def iteration(t, s, stage, fetch):
            if stage:
                s2 = (s + 2) % NBUF
                pack_wait(s2)
                unpack(s2)
                if fetch:
                    pack_start(t + 4, (s + 1) % NBUF)
                gather_start(s2)
            gather_wait(s)
            compute(s)
            scatter_sync(s)

        pack_start(0, 0)
        pack_start(1, 1)
        pack_wait(0)
        unpack(0)
        gather_start(0)
        pack_start(2, 2)
        pack_wait(1)
        unpack(1)
        gather_start(1)
        pack_start(3, 0)

        def trio(i, _):
            for j in range(3):
                t = 3 * i + j
                iteration(t, j, True, True)
            return 0
        lax.fori_loop(0, (n_chunks - 6) // 3, trio, 0)

        nt = n_chunks
        iteration(nt - 6, (nt - 6) % 3, True, True)
        iteration(nt - 5, (nt - 5) % 3, True, True)
        iteration(nt - 4, (nt - 4) % 3, True, False)
        iteration(nt - 3, (nt - 3) % 3, True, False)
        iteration(nt - 2, (nt - 2) % 3, False, False)
        iteration(nt - 1, (nt - 1) % 3, False, False)
        plsc.subcore_barrier()

        pltpu.sync_copy(acc.at[pl.ds(r0, rows_per_tile)],
                        out_hbm.at[cid, pl.ds(r0, rows_per_tile)])

    return k(x, pack)


def _tc_reduce(partials):
    br = 632

    def add_body(p_ref, o_ref):
        o_ref[...] = p_ref[0] + p_ref[1]

    return pl.pallas_call(
        add_body,
        grid=(N_PAD // br,),
        in_specs=[pl.BlockSpec((2, br, D_FEAT), lambda i: (0, i, 0))],
        out_specs=pl.BlockSpec((br, D_FEAT), lambda i: (i, 0)),
        out_shape=jax.ShapeDtypeStruct((N_PAD, D_FEAT), jnp.float32),
    )(partials)


def kernel(x, edge_index, edge_attr):
    row = edge_index[0]
    col = edge_index[1]
    n_edges = row.shape[0]
    gran = NW * 6 * CHUNK
    e_per_w = ((n_edges + gran - 1) // gran) * 6 * CHUNK
    n_chunks = e_per_w // CHUNK
    e_pad = e_per_w * NW
    pad = e_pad - n_edges
    row_p = jnp.concatenate([row, jnp.zeros((pad,), jnp.int32)])
    col_p = jnp.concatenate([col, jnp.zeros((pad,), jnp.int32)])
    attr_p = jnp.concatenate(
        [edge_attr, jnp.full((pad,), 1e30, jnp.float32)])
    t_total = e_pad // CHUNK
    pack = jnp.concatenate(
        [
            row_p.astype(jnp.float32).reshape(t_total, 1, CHUNK),
            col_p.astype(jnp.float32).reshape(t_total, 1, CHUNK),
            attr_p.reshape(t_total, 1, CHUNK),
            jnp.zeros((t_total, 5, CHUNK), jnp.float32),
        ],
        axis=1,
    )
    partials = _sc_partials(x, pack, n_chunks)
    return _tc_reduce(partials)[:N_NODES]


# speedup vs baseline: 1.0010x; 1.0010x over previous
import functools

import jax
import jax.numpy as jnp
from jax import lax
from jax.experimental import pallas as pl
from jax.experimental.pallas import tpu as pltpu
from jax.experimental.pallas import tpu_sc as plsc

N_NODES = 10000
N_PAD = 10112
D_FEAT = 128
NC = 2
NS = 16
NW = NC * NS
L = 16
CHUNK = 112
NBUF = 3


def _sc_partials(x, pack, n_chunks):
    rows_per_tile = N_PAD // NS

    mesh = plsc.VectorSubcoreMesh(core_axis_name="c", subcore_axis_name="s")

    @functools.partial(
        pl.kernel,
        out_type=jax.ShapeDtypeStruct((NC, N_PAD, D_FEAT), jnp.float32),
        mesh=mesh,
        scratch_types=[
            pltpu.VMEM((NBUF, 8, CHUNK), jnp.float32),
            pltpu.VMEM((CHUNK + L,), jnp.float32),
            pltpu.VMEM((CHUNK + L,), jnp.float32),
            pltpu.VMEM((CHUNK + L,), jnp.float32),
            pltpu.VMEM((CHUNK,), jnp.int32),
            pltpu.VMEM((CHUNK,), jnp.int32),
            pltpu.VMEM((CHUNK,), jnp.int32),
            pltpu.VMEM((CHUNK,), jnp.int32),
            pltpu.VMEM((CHUNK,), jnp.int32),
            pltpu.VMEM((CHUNK,), jnp.int32),
            pltpu.VMEM((NBUF, CHUNK, D_FEAT), jnp.float32),
            pltpu.VMEM_SHARED((N_PAD, D_FEAT), jnp.float32),
            pltpu.SemaphoreType.DMA,
            pltpu.SemaphoreType.DMA,
            pltpu.SemaphoreType.DMA,
            pltpu.SemaphoreType.DMA,
            pltpu.SemaphoreType.DMA,
            pltpu.SemaphoreType.DMA,
        ],
    )
    def k(x_hbm, pack_hbm, out_hbm, pack_v, b0, b1, b2, ri0, ri1, ri2,
          ci0, ci1, ci2, rows_v, acc, sg0, sg1, sg2, sp0, sp1, sp2):
        sgs = [sg0, sg1, sg2]
        sps = [sp0, sp1, sp2]
        bvs = [b0, b1, b2]
        ris = [ri0, ri1, ri2]
        cis = [ci0, ci1, ci2]
        cid = lax.axis_index("c")
        sid = lax.axis_index("s")
        wid = sid * NC + cid
        chunk0 = wid * n_chunks

        zeros = jnp.zeros((L,), jnp.float32)
        def zrow(r, _):
            for g in range(D_FEAT // L):
                rows_v[0, r, pl.ds(g * L, L)] = zeros
            return 0
        lax.fori_loop(0, CHUNK, zrow, 0)
        r0 = sid * rows_per_tile
        for piece in range(5):
            pltpu.sync_copy(rows_v.at[0],
                            acc.at[pl.ds(r0 + piece * CHUNK, CHUNK)])
        rem = rows_per_tile - 5 * CHUNK
        pltpu.sync_copy(rows_v.at[0, pl.ds(0, rem)],
                        acc.at[pl.ds(r0 + 5 * CHUNK, rem)])
        plsc.subcore_barrier()

        def pack_start(t, s):
            pltpu.async_copy(pack_hbm.at[chunk0 + t], pack_v.at[s], sps[s])

        def pack_wait(s):
            pltpu.make_async_copy(pack_hbm.at[chunk0], pack_v.at[s],
                                  sps[s]).wait()

        def unpack(s):
            for g in range(CHUNK // L):
                sl = pl.ds(g * L, L)
                ris[s][sl] = pack_v[s, 0, sl].astype(jnp.int32)
                cis[s][sl] = pack_v[s, 1, sl].astype(jnp.int32)
                bvs[s][sl] = pack_v[s, 2, sl]

        def gather_start(s):
            pltpu.async_copy(x_hbm.at[cis[s]], rows_v.at[s], sgs[s])

        def gather_wait(s):
            pltpu.make_async_copy(x_hbm.at[cis[s]],
                                  rows_v.at[s], sgs[s]).wait()

        def scatter_sync(s):
            pltpu.sync_copy(rows_v.at[s], acc.at[ris[s]], add=True)

        def compute(s):
            bv = bvs[s]
            for g in range(CHUNK // L):
                bv[pl.ds(g * L, L)] = jnp.exp(-bv[pl.ds(g * L, L)])
            def edge_body(e, _):
                b = bv[pl.ds(e, L)][0]
                for g in range(D_FEAT // L):
                    rows_v[s, e, pl.ds(g * L, L)] = (
                        rows_v[s, e, pl.ds(g * L, L)] * b)
                return 0
            lax.fori_loop(0, CHUNK, edge_body, 0, unroll=8)

        def iteration(t, s, stage, fetch):
            if stage:
                s2 = (s + 2) % NBUF
                pack_wait(s2)
                unpack(s2)
                if fetch:
                    pack_start(t + 4, (s + 1) % NBUF)
                gather_start(s2)
            gather_wait(s)
            compute(s)
            scatter_sync(s)

        pack_start(0, 0)
        pack_start(1, 1)
        pack_wait(0)
        unpack(0)
        gather_start(0)
        pack_start(2, 2)
        pack_wait(1)
        unpack(1)
        gather_start(1)
        pack_start(3, 0)

        def trio(i, _):
            for j in range(3):
                t = 3 * i + j
                iteration(t, j, True, True)
            return 0
        lax.fori_loop(0, (n_chunks - 6) // 3, trio, 0)

        nt = n_chunks
        iteration(nt - 6, (nt - 6) % 3, True, True)
        iteration(nt - 5, (nt - 5) % 3, True, True)
        iteration(nt - 4, (nt - 4) % 3, True, False)
        iteration(nt - 3, (nt - 3) % 3, True, False)
        iteration(nt - 2, (nt - 2) % 3, False, False)
        iteration(nt - 1, (nt - 1) % 3, False, False)
        plsc.subcore_barrier()

        pltpu.sync_copy(acc.at[pl.ds(r0, rows_per_tile)],
                        out_hbm.at[cid, pl.ds(r0, rows_per_tile)])

    return k(x, pack)


def _tc_reduce(partials):
    br = 632

    def add_body(p_ref, o_ref):
        o_ref[...] = p_ref[0] + p_ref[1]

    return pl.pallas_call(
        add_body,
        grid=(N_PAD // br,),
        in_specs=[pl.BlockSpec((2, br, D_FEAT), lambda i: (0, i, 0))],
        out_specs=pl.BlockSpec((br, D_FEAT), lambda i: (i, 0)),
        out_shape=jax.ShapeDtypeStruct((N_PAD, D_FEAT), jnp.float32),
    )(partials)


def kernel(x, edge_index, edge_attr):
    row = edge_index[0]
    col = edge_index[1]
    n_edges = row.shape[0]
    gran = NW * 6 * CHUNK
    e_per_w = ((n_edges + gran - 1) // gran) * 6 * CHUNK
    n_chunks = e_per_w // CHUNK
    e_pad = e_per_w * NW
    pad = e_pad - n_edges
    row_p = jnp.concatenate([row, jnp.zeros((pad,), jnp.int32)])
    col_p = jnp.concatenate([col, jnp.zeros((pad,), jnp.int32)])
    attr_p = jnp.concatenate(
        [edge_attr, jnp.full((pad,), 1e30, jnp.float32)])
    t_total = e_pad // CHUNK
    pack = jnp.concatenate(
        [
            row_p.astype(jnp.float32).reshape(t_total, 1, CHUNK),
            col_p.astype(jnp.float32).reshape(t_total, 1, CHUNK),
            attr_p.reshape(t_total, 1, CHUNK),
            jnp.zeros((t_total, 5, CHUNK), jnp.float32),
        ],
        axis=1,
    )
    partials = _sc_partials(x, pack, n_chunks)
    return _tc_reduce(partials)[:N_NODES]
